# Initial kernel scaffold; baseline (speedup 1.0000x reference)
import jax
import jax.numpy as jnp
from jax.experimental import pallas as pl


def kernel(token_ids, weight):
    raise NotImplementedError("write your pallas kernel here")


# speedup vs baseline: 4.8126x; 4.8126x over previous
import functools

import jax
import jax.numpy as jnp
from jax import lax
from jax.experimental import pallas as pl
from jax.experimental.pallas import tpu as pltpu
from jax.experimental.pallas import tpu_sc as plsc

NUM_ROWS = 1_000_000
DIM = 32
NC = 2
NS = 16
NW = NC * NS
VEC = 128
CV = 8
CHUNK = CV * VEC


def _body(idx_hbm, table_hbm, out_hbm, idx_v, rows_v, sem):
    steps = idx_hbm.shape[1]
    wid = lax.axis_index("s") * NC + lax.axis_index("c")
    base = wid * steps * CHUNK

    def step(j, carry):
        pltpu.sync_copy(idx_hbm.at[wid, j], idx_v)
        cps = [
            pltpu.async_copy(
                table_hbm.at[idx_v.at[s]],
                rows_v.at[pl.ds(s * VEC, VEC)],
                sem,
            )
            for s in range(CV)
        ]
        for c in cps:
            c.wait()
        pltpu.sync_copy(rows_v, out_hbm.at[pl.ds(base + j * CHUNK, CHUNK)])
        return carry

    lax.fori_loop(0, steps, step, 0)


def kernel(token_ids, weight):
    B, S = token_ids.shape
    total = B * S
    assert total % (NW * CHUNK) == 0
    steps = total // (NW * CHUNK)
    idx = token_ids.reshape(NW, steps, CV, VEC)

    grab = pl.kernel(
        _body,
        out_type=jax.ShapeDtypeStruct((total, DIM), jnp.float32),
        mesh=plsc.VectorSubcoreMesh(
            core_axis_name="c", subcore_axis_name="s",
            num_cores=NC, num_subcores=NS,
        ),
        scratch_types=[
            pltpu.VMEM((CV, VEC), jnp.int32),
            pltpu.VMEM((CHUNK, DIM), jnp.float32),
            pltpu.SemaphoreType.DMA,
        ],
        compiler_params=pltpu.CompilerParams(use_tc_tiling_on_sc=False),
    )
    out = grab(idx, weight)
    return out.reshape(B, S, DIM)


# speedup vs baseline: 4.9584x; 1.0303x over previous
import jax
import jax.numpy as jnp
from jax import lax
from jax.experimental import pallas as pl
from jax.experimental.pallas import tpu as pltpu
from jax.experimental.pallas import tpu_sc as plsc

NUM_ROWS = 1_000_000
DIM = 32
NC = 2
NS = 16
NW = NC * NS
VEC = 128
CV = 8
CHUNK = CV * VEC


def _body(idx_hbm, table_hbm, out_hbm, idx_v, rows_v,
          gsem0, gsem1, wsem0, wsem1):
    steps = idx_hbm.shape[1]
    assert steps >= 4 and steps % 2 == 0
    gsems = (gsem0, gsem1)
    wsems = (wsem0, wsem1)
    wid = lax.axis_index("s") * NC + lax.axis_index("c")
    base = wid * steps * CHUNK

    def fire(j, b):
        pltpu.sync_copy(idx_hbm.at[wid, j], idx_v.at[b])
        for s in range(CV):
            pltpu.async_copy(
                table_hbm.at[idx_v.at[b, s]],
                rows_v.at[b, pl.ds(s * VEC, VEC)],
                gsems[b],
            )

    def drain_gather(b):
        pltpu.make_async_copy(
            out_hbm.at[pl.ds(base, CHUNK)], rows_v.at[b], gsems[b]
        ).wait()

    def write(j, b):
        pltpu.async_copy(
            rows_v.at[b], out_hbm.at[pl.ds(base + j * CHUNK, CHUNK)], wsems[b]
        )

    def drain_write(b):
        pltpu.make_async_copy(
            rows_v.at[b], out_hbm.at[pl.ds(base, CHUNK)], wsems[b]
        ).wait()

    fire(0, 0)
    fire(1, 1)
    drain_gather(0)
    write(0, 0)

    def body(g, carry):
        j = 2 * g + 1
        drain_write(0)
        fire(j + 1, 0)
        drain_gather(1)
        write(j, 1)
        jj = j + 1
        drain_write(1)
        fire(jj + 1, 1)
        drain_gather(0)
        write(jj, 0)
        return carry

    lax.fori_loop(0, (steps - 2) // 2, body, 0)

    drain_gather(1)
    write(steps - 1, 1)
    drain_write(0)
    drain_write(1)


def kernel(token_ids, weight):
    B, S = token_ids.shape
    total = B * S
    assert total % (NW * CHUNK) == 0
    steps = total // (NW * CHUNK)
    idx = token_ids.reshape(NW, steps, CV, VEC)

    grab = pl.kernel(
        _body,
        out_type=jax.ShapeDtypeStruct((total, DIM), jnp.float32),
        mesh=plsc.VectorSubcoreMesh(
            core_axis_name="c", subcore_axis_name="s",
            num_cores=NC, num_subcores=NS,
        ),
        scratch_types=[
            pltpu.VMEM((2, CV, VEC), jnp.int32),
            pltpu.VMEM((2, CHUNK, DIM), jnp.float32),
            pltpu.SemaphoreType.DMA,
            pltpu.SemaphoreType.DMA,
            pltpu.SemaphoreType.DMA,
            pltpu.SemaphoreType.DMA,
        ],
        compiler_params=pltpu.CompilerParams(use_tc_tiling_on_sc=False),
    )
    out = grab(idx, weight)
    return out.reshape(B, S, DIM)


# speedup vs baseline: 4.9701x; 1.0024x over previous
import jax
import jax.numpy as jnp
from jax import lax
from jax.experimental import pallas as pl
from jax.experimental.pallas import tpu as pltpu
from jax.experimental.pallas import tpu_sc as plsc

NUM_ROWS = 1_000_000
DIM = 32
NC = 2
NS = 16
NW = NC * NS
VEC = 128
CV = 8
CHUNK = CV * VEC
NBUF = 3


def _body(idx_hbm, table_hbm, out_hbm, idx_v, rows_v, gsem, wsem):
    steps = idx_hbm.shape[1]
    assert steps >= 2 * NBUF
    wid = lax.axis_index("s") * NC + lax.axis_index("c")
    base = wid * steps * CHUNK

    def fire(j, b):
        pltpu.sync_copy(idx_hbm.at[wid, j], idx_v.at[b])
        for s in range(CV):
            pltpu.async_copy(
                table_hbm.at[idx_v.at[b, s]],
                rows_v.at[b, pl.ds(s * VEC, VEC)],
                gsem.at[b],
            )

    def retire(j, b):
        pltpu.make_async_copy(
            out_hbm.at[pl.ds(base, CHUNK)], rows_v.at[b], gsem.at[b]
        ).wait()
        pltpu.async_copy(
            rows_v.at[b], out_hbm.at[pl.ds(base + j * CHUNK, CHUNK)],
            wsem.at[b],
        )

    def drain_write(b):
        pltpu.make_async_copy(
            rows_v.at[b], out_hbm.at[pl.ds(base, CHUNK)], wsem.at[b]
        ).wait()

    for j in range(NBUF - 1):
        fire(j, j)

    def body(j, carry):
        b = j % NBUF

        @pl.when(j >= NBUF)
        def _():
            drain_write(b)

        fire(j, b)
        jr = j - (NBUF - 1)
        retire(jr, jr % NBUF)
        return carry

    lax.fori_loop(NBUF - 1, steps, body, 0)

    for jr in range(steps - NBUF + 1, steps):
        retire(jr, jr % NBUF)
    for b in range(NBUF):
        drain_write(b)


def kernel(token_ids, weight):
    B, S = token_ids.shape
    total = B * S
    assert total % (NW * CHUNK) == 0
    steps = total // (NW * CHUNK)
    idx = token_ids.reshape(NW, steps, CV, VEC)

    grab = pl.kernel(
        _body,
        out_type=jax.ShapeDtypeStruct((total, DIM), jnp.float32),
        mesh=plsc.VectorSubcoreMesh(
            core_axis_name="c", subcore_axis_name="s",
            num_cores=NC, num_subcores=NS,
        ),
        scratch_types=[
            pltpu.VMEM((NBUF, CV, VEC), jnp.int32),
            pltpu.VMEM((NBUF, CHUNK, DIM), jnp.float32),
            pltpu.SemaphoreType.DMA((NBUF,)),
            pltpu.SemaphoreType.DMA((NBUF,)),
        ],
        compiler_params=pltpu.CompilerParams(use_tc_tiling_on_sc=False),
    )
    out = grab(idx, weight)
    return out.reshape(B, S, DIM)


# speedup vs baseline: 5.4464x; 1.0958x over previous
import jax
import jax.numpy as jnp
from jax import lax
from jax.experimental import pallas as pl
from jax.experimental.pallas import tpu as pltpu
from jax.experimental.pallas import tpu_sc as plsc

NUM_ROWS = 1_000_000
DIM = 32
NC = 2
NS = 16
NW = NC * NS
VEC = 128
SB = 8
CHUNK = SB * VEC
NBUF = 3


def _body(idx_hbm, table_hbm, out_hbm, idx_v, rows_v, gsem, wsem):
    n_chunks = idx_hbm.shape[0]
    per_w = n_chunks // NW
    wid = lax.axis_index("s") * NC + lax.axis_index("c")

    def fire(j, b):
        cblk = wid * per_w + j
        pltpu.sync_copy(idx_hbm.at[cblk], idx_v.at[b])
        for r in range(SB):
            pltpu.async_copy(
                table_hbm.at[idx_v.at[b, r]],
                rows_v.at[b, pl.ds(r * VEC, VEC)],
                gsem.at[b],
            )

    def retire(j, b):
        pltpu.make_async_copy(
            out_hbm.at[0, pl.ds(0, CHUNK)], rows_v.at[b], gsem.at[b]
        ).wait()
        cblk = wid * per_w + j
        rb = cblk // 128
        b0 = (cblk % 128) * VEC
        for r in range(SB):
            pltpu.async_copy(
                rows_v.at[b, pl.ds(r * VEC, VEC)],
                out_hbm.at[8 * rb + r, pl.ds(b0, VEC)],
                wsem.at[b],
            )

    def drain_write(b):
        pltpu.make_async_copy(
            rows_v.at[b], out_hbm.at[0, pl.ds(0, CHUNK)], wsem.at[b]
        ).wait()

    for j in range(NBUF - 1):
        fire(j, j)

    def body(j, carry):
        b = j % NBUF

        @pl.when(j >= NBUF)
        def _():
            drain_write(b)

        fire(j, b)
        jr = j - (NBUF - 1)
        retire(jr, jr % NBUF)
        return carry

    lax.fori_loop(NBUF - 1, per_w, body, 0)

    for jr in range(per_w - NBUF + 1, per_w):
        retire(jr, jr % NBUF)
    for b in range(NBUF):
        drain_write(b)


def kernel(token_ids, weight):
    B, S = token_ids.shape
    assert B % VEC == 0 and S % SB == 0
    n_chunks = (B // VEC) * (S // SB)
    assert n_chunks % NW == 0
    idx = (
        token_ids.T.reshape(S // SB, SB, B // VEC, VEC)
        .transpose(0, 2, 1, 3)
        .reshape(n_chunks, SB, VEC)
    )

    grab = pl.kernel(
        _body,
        out_type=jax.ShapeDtypeStruct((S, B, DIM), jnp.float32),
        mesh=plsc.VectorSubcoreMesh(
            core_axis_name="c", subcore_axis_name="s",
            num_cores=NC, num_subcores=NS,
        ),
        scratch_types=[
            pltpu.VMEM((NBUF, SB, VEC), jnp.int32),
            pltpu.VMEM((NBUF, CHUNK, DIM), jnp.float32),
            pltpu.SemaphoreType.DMA((NBUF,)),
            pltpu.SemaphoreType.DMA((NBUF,)),
        ],
        compiler_params=pltpu.CompilerParams(use_tc_tiling_on_sc=False),
    )
    out = grab(idx, weight)
    return out.transpose(1, 0, 2)
